# baseline (speedup 1.0000x reference)
import functools

import jax
import jax.numpy as jnp
from jax import lax
from jax.experimental import pallas as pl
from jax.experimental.pallas import tpu as pltpu
from jax.experimental.pallas import tpu_sc as plsc

OUT_CHANNELS = 32
N_CAT = 26
N_NUM = 13
VOCAB = 100000
BATCH = 16384
N_COLS = 39
TAB_ROWS = N_CAT * (VOCAB + 1)

NUM_CORES = 2
NUM_SUBCORES = 16
NW = NUM_CORES * NUM_SUBCORES
ROWS_PER_W = BATCH // NW
CB = 64
N_CHUNKS = ROWS_PER_W // CB


def _body(x_hbm, tab_hbm, out_hbm, x_v, idx_v, obuf, sem):
    wid = lax.axis_index("s") * NUM_CORES + lax.axis_index("c")

    iota = lax.iota(jnp.int32, 16)
    off_a = iota * (VOCAB + 1)
    off_b = (iota + 10) * (VOCAB + 1)

    def chunk_body(s, carry):
        base = wid * ROWS_PER_W + s * CB
        pltpu.sync_copy(x_hbm.at[pl.ds(base, CB)], x_v)

        def idx_body(b, c):
            r0 = x_v[b, pl.ds(13, 16)].astype(jnp.int32) + off_a
            r1 = x_v[b, pl.ds(23, 16)].astype(jnp.int32) + off_b
            idx_v[b, pl.ds(0, 16)] = r0
            idx_v[b, pl.ds(10, 16)] = r1
            return c

        lax.fori_loop(0, CB, idx_body, 0)

        def fire_body(b, c):
            pltpu.async_copy(
                tab_hbm.at[idx_v.at[b]], obuf.at[b, pl.ds(N_NUM, N_CAT)], sem
            )
            return c

        lax.fori_loop(0, CB, fire_body, 0)

        def num_body(b, c):
            row = x_v[b, pl.ds(0, 16)]
            for j in range(N_NUM):
                spl = jnp.full((16,), row[j], jnp.float32)
                obuf[b, j, pl.ds(0, 16)] = spl
                obuf[b, j, pl.ds(16, 16)] = spl
            return c

        lax.fori_loop(0, CB, num_body, 0)

        def drain_body(b, c):
            pltpu.make_async_copy(
                tab_hbm.at[idx_v.at[b]], obuf.at[b, pl.ds(N_NUM, N_CAT)], sem
            ).wait()
            return c

        lax.fori_loop(0, CB, drain_body, 0)

        pltpu.sync_copy(obuf, out_hbm.at[pl.ds(base, CB)])
        return carry

    lax.fori_loop(0, N_CHUNKS, chunk_body, 0)


_mesh = plsc.VectorSubcoreMesh(
    core_axis_name="c", subcore_axis_name="s",
    num_cores=NUM_CORES, num_subcores=NUM_SUBCORES,
)

_encode = pl.kernel(
    _body,
    out_type=jax.ShapeDtypeStruct((BATCH, N_COLS, OUT_CHANNELS), jnp.float32),
    mesh=_mesh,
    scratch_types=[
        pltpu.VMEM((CB, N_COLS), jnp.float32),
        pltpu.VMEM((CB, N_CAT), jnp.int32),
        pltpu.VMEM((CB, N_COLS, OUT_CHANNELS), jnp.float32),
        pltpu.SemaphoreType.DMA,
    ],
    compiler_params=pltpu.CompilerParams(use_tc_tiling_on_sc=False),
)


@jax.jit
def kernel(x, tables):
    tab_flat = tables.reshape(TAB_ROWS, OUT_CHANNELS)
    return _encode(x, tab_flat)
